# Initial kernel scaffold; baseline (speedup 1.0000x reference)
import jax
import jax.numpy as jnp
from jax.experimental import pallas as pl


def kernel(x, t, pe):
    raise NotImplementedError("write your pallas kernel here")


# speedup vs baseline: 4.3478x; 4.3478x over previous
import functools

import jax
import jax.numpy as jnp
from jax import lax
from jax.experimental import pallas as pl
from jax.experimental.pallas import tpu as pltpu
from jax.experimental.pallas import tpu_sc as plsc

_DIM = 64
_NC = 2
_NS = 16
_NW = _NC * _NS

_CHUNK = 256
_IDX_PER_STREAM = 128
_STREAMS = _CHUNK // _IDX_PER_STREAM
_LANES = 16
_ROW_UNROLL = 8


def _pe_concat_kernel(n_iters, x_ref, t_ref, pe2_ref, out_ref,
                      idx_v, outv, xv, sem):
    wid = lax.axis_index("s") * _NC + lax.axis_index("c")
    per_worker = n_iters * _CHUNK

    def body(it, _):
        base = wid * per_worker + it * _CHUNK
        pltpu.sync_copy(t_ref.at[pl.ds(base, _CHUNK)], idx_v)
        copies = []
        for j in range(_STREAMS):
            copies.append(pltpu.async_copy(
                pe2_ref.at[idx_v.at[pl.ds(j * _IDX_PER_STREAM,
                                          _IDX_PER_STREAM)]],
                outv.at[pl.ds(j * _IDX_PER_STREAM, _IDX_PER_STREAM)],
                sem))
        copies.append(pltpu.async_copy(
            x_ref.at[pl.ds(base, _CHUNK)], xv, sem))
        for c in copies:
            c.wait()

        def vbody(i, _):
            r0 = i * _ROW_UNROLL
            for u in range(_ROW_UNROLL):
                for c in range(_DIM // _LANES):
                    outv[r0 + u, pl.ds(c * _LANES, _LANES)] = (
                        xv[r0 + u, pl.ds(c * _LANES, _LANES)])
            return ()

        lax.fori_loop(0, _CHUNK // _ROW_UNROLL, vbody, ())

        pltpu.sync_copy(outv, out_ref.at[pl.ds(base, _CHUNK)])
        return ()

    lax.fori_loop(0, n_iters, body, ())


def kernel(x, t, pe):
    batch, seq, dim = x.shape
    n = batch * seq
    assert n % (_NW * _CHUNK) == 0
    n_iters = n // (_NW * _CHUNK)

    x2 = x.reshape(n, dim)
    t1 = t.reshape(n)
    pe2 = jnp.concatenate([jnp.zeros_like(pe), pe], axis=1)

    mesh = plsc.VectorSubcoreMesh(core_axis_name="c", subcore_axis_name="s")
    out = pl.kernel(
        functools.partial(_pe_concat_kernel, n_iters),
        out_type=jax.ShapeDtypeStruct((n, 2 * dim), jnp.float32),
        mesh=mesh,
        scratch_types=[
            pltpu.VMEM((_CHUNK,), jnp.int32),
            pltpu.VMEM((_CHUNK, 2 * dim), jnp.float32),
            pltpu.VMEM((_CHUNK, dim), jnp.float32),
            pltpu.SemaphoreType.DMA,
        ],
    )(x2, t1, pe2)
    return out.reshape(batch, seq, 2 * dim)
